# baseline (speedup 1.0000x reference)
import functools

import jax
import jax.numpy as jnp
from jax import lax
from jax.experimental import pallas as pl
from jax.experimental.pallas import tpu as pltpu
from jax.experimental.pallas import tpu_sc as plsc

N_NODES = 10000
N_EDGES = 160000
HID = 256
HALF = 128
NUM_SUBCORES = 16
EDGES_PER_SUB = N_EDGES // NUM_SUBCORES
CHUNK = 50
NCHUNK = EDGES_PER_SUB // CHUNK
NCHUNK_B = 20
NSUPER = NCHUNK // NCHUNK_B
NBUF = 5
ROW_STRIPE = 624
ROW_TAIL = N_NODES - ROW_STRIPE * NUM_SUBCORES


def _sc_aggregate(h0, h1, src, dst, zeros):
    mesh = plsc.VectorSubcoreMesh(core_axis_name="c", subcore_axis_name="s")

    @functools.partial(
        pl.kernel,
        out_type=[jax.ShapeDtypeStruct((N_NODES, HALF), jnp.float32)] * 2,
        mesh=mesh,
        scratch_types=(
            [pltpu.VMEM((NCHUNK_B, CHUNK), jnp.int32)] * 4
            + [pltpu.VMEM((CHUNK, HALF), jnp.float32)] * NBUF
            + [pltpu.VMEM_SHARED((N_NODES, HALF), jnp.float32)]
            + [pltpu.SemaphoreType.DMA] * (2 * NBUF + 2)
        ),
    )
    def agg_kernel(h0_hbm, h1_hbm, src_hbm, dst_hbm, zeros_hbm,
                   out0_hbm, out1_hbm, src0_v, dst0_v, src1_v, dst1_v, *rest):
        rows = list(rest[:NBUF])
        acc_sh = rest[NBUF]
        gsem = list(rest[NBUF + 1:NBUF + 1 + NBUF])
        ssem = list(rest[NBUF + 1 + NBUF:NBUF + 1 + 2 * NBUF])
        isem = list(rest[NBUF + 1 + 2 * NBUF:NBUF + 3 + 2 * NBUF])
        c = lax.axis_index("c")
        s = lax.axis_index("s")
        row0 = s * ROW_STRIPE
        tail0 = NUM_SUBCORES * ROW_STRIPE
        pltpu.sync_copy(zeros_hbm.at[pl.ds(row0, ROW_STRIPE)],
                        acc_sh.at[pl.ds(row0, ROW_STRIPE)])

        @pl.when(s == NUM_SUBCORES - 1)
        def _():
            pltpu.sync_copy(zeros_hbm.at[pl.ds(tail0, ROW_TAIL)],
                            acc_sh.at[pl.ds(tail0, ROW_TAIL)])

        plsc.subcore_barrier()

        def idx_start(b, sv, dv, sm):
            pltpu.async_copy(src_hbm.at[s, b], sv, sm)
            pltpu.async_copy(dst_hbm.at[s, b], dv, sm)

        def idx_wait(b, sv, dv, sm):
            pltpu.make_async_copy(src_hbm.at[s, b], sv, sm).wait()
            pltpu.make_async_copy(dst_hbm.at[s, b], dv, sm).wait()

        def edge_loop(h_hbm):
            def start_gather(src_v, j, k):
                pltpu.async_copy(h_hbm.at[src_v.at[j]], rows[k], gsem[k])

            def wait_gather(src_v, j, k):
                pltpu.make_async_copy(h_hbm.at[src_v.at[j]], rows[k],
                                      gsem[k]).wait()

            def start_scatter(dst_v, j, k):
                pltpu.async_copy(rows[k], acc_sh.at[dst_v.at[j]],
                                 ssem[k], add=True)

            def wait_scatter(dst_v, j, k):
                pltpu.make_async_copy(rows[k], acc_sh.at[dst_v.at[j]],
                                      ssem[k]).wait()

            def round_(src_v, dst_v, j0, refill):
                for k in range(NBUF):
                    wait_gather(src_v, j0 + k, k)
                    start_scatter(dst_v, j0 + k, k)
                for k in range(NBUF):
                    wait_scatter(dst_v, j0 + k, k)
                    refill(k)

            def super_block(src_v, dst_v, nxt_src_v, more):
                @pl.loop(0, NCHUNK_B - NBUF, step=NBUF)
                def _(j0):
                    round_(src_v, dst_v, j0,
                           lambda k: start_gather(src_v, j0 + NBUF + k, k))

                def cross_refill(k):
                    @pl.when(more)
                    def _():
                        start_gather(nxt_src_v, k, k)

                round_(src_v, dst_v, NCHUNK_B - NBUF, cross_refill)

            idx_start(0, src0_v, dst0_v, isem[0])
            idx_wait(0, src0_v, dst0_v, isem[0])
            for k in range(NBUF):
                start_gather(src0_v, k, k)
            idx_start(1, src1_v, dst1_v, isem[1])

            @pl.loop(0, NSUPER, step=2)
            def _(b):
                idx_wait(b + 1, src1_v, dst1_v, isem[1])
                super_block(src0_v, dst0_v, src1_v, b + 1 < NSUPER)

                @pl.when(b + 2 < NSUPER)
                def _():
                    idx_start(b + 2, src0_v, dst0_v, isem[0])
                    idx_wait(b + 2, src0_v, dst0_v, isem[0])

                super_block(src1_v, dst1_v, src0_v, b + 2 < NSUPER)

                @pl.when(b + 3 < NSUPER)
                def _():
                    idx_start(b + 3, src1_v, dst1_v, isem[1])

        @pl.when(c == 0)
        def _():
            edge_loop(h0_hbm)

        @pl.when(c == 1)
        def _():
            edge_loop(h1_hbm)

        plsc.subcore_barrier()

        def writeback(out_hbm):
            pltpu.sync_copy(acc_sh.at[pl.ds(row0, ROW_STRIPE)],
                            out_hbm.at[pl.ds(row0, ROW_STRIPE)])

            @pl.when(s == NUM_SUBCORES - 1)
            def _():
                pltpu.sync_copy(acc_sh.at[pl.ds(tail0, ROW_TAIL)],
                                out_hbm.at[pl.ds(tail0, ROW_TAIL)])

        @pl.when(c == 0)
        def _():
            writeback(out0_hbm)

        @pl.when(c == 1)
        def _():
            writeback(out1_hbm)

    return agg_kernel(h0, h1, src, dst, zeros)


_BLK = 1000


def _tc_layer(agg0, agg1, W1, b1r):
    def body(a0_ref, a1_ref, w_ref, b_ref, h0_ref, h1_ref):
        y = jnp.dot(a0_ref[...], w_ref[:HALF, :],
                    preferred_element_type=jnp.float32,
                    precision=lax.Precision.HIGHEST)
        y = y + jnp.dot(a1_ref[...], w_ref[HALF:, :],
                        preferred_element_type=jnp.float32,
                        precision=lax.Precision.HIGHEST)
        y = jnp.maximum(y + b_ref[...], 0.0)
        h0_ref[...] = y[:, :HALF]
        h1_ref[...] = y[:, HALF:]

    return pl.pallas_call(
        body,
        grid=(N_NODES // _BLK,),
        in_specs=[
            pl.BlockSpec((_BLK, HALF), lambda i: (i, 0)),
            pl.BlockSpec((_BLK, HALF), lambda i: (i, 0)),
            pl.BlockSpec((HID, HID), lambda i: (0, 0)),
            pl.BlockSpec((1, HID), lambda i: (0, 0)),
        ],
        out_specs=[
            pl.BlockSpec((_BLK, HALF), lambda i: (i, 0)),
            pl.BlockSpec((_BLK, HALF), lambda i: (i, 0)),
        ],
        out_shape=[jax.ShapeDtypeStruct((N_NODES, HALF), jnp.float32)] * 2,
    )(agg0, agg1, W1, b1r)


def _tc_final(agg0, agg1, W1, b1r, W_out, b_outr):
    def body(a0_ref, a1_ref, w_ref, b_ref, wo_ref, bo_ref, out_ref):
        y = jnp.dot(a0_ref[...], w_ref[:HALF, :],
                    preferred_element_type=jnp.float32,
                    precision=lax.Precision.HIGHEST)
        y = y + jnp.dot(a1_ref[...], w_ref[HALF:, :],
                        preferred_element_type=jnp.float32,
                        precision=lax.Precision.HIGHEST)
        y = jnp.maximum(y + b_ref[...], 0.0)
        out_ref[...] = jnp.dot(y, wo_ref[...],
                               preferred_element_type=jnp.float32,
                               precision=lax.Precision.HIGHEST) + bo_ref[...]

    return pl.pallas_call(
        body,
        grid=(N_NODES // _BLK,),
        in_specs=[
            pl.BlockSpec((_BLK, HALF), lambda i: (i, 0)),
            pl.BlockSpec((_BLK, HALF), lambda i: (i, 0)),
            pl.BlockSpec((HID, HID), lambda i: (0, 0)),
            pl.BlockSpec((1, HID), lambda i: (0, 0)),
            pl.BlockSpec((HID, HID), lambda i: (0, 0)),
            pl.BlockSpec((1, HID), lambda i: (0, 0)),
        ],
        out_specs=pl.BlockSpec((_BLK, HID), lambda i: (i, 0)),
        out_shape=jax.ShapeDtypeStruct((N_NODES, HID), jnp.float32),
    )(agg0, agg1, W1, b1r, W_out, b_outr)


def kernel(features, edge_index, W1, b1, W_out, b_out):
    eidx = edge_index.astype(jnp.int32)
    src = eidx[0].reshape(NUM_SUBCORES, NSUPER, NCHUNK_B, CHUNK)
    dst = eidx[1].reshape(NUM_SUBCORES, NSUPER, NCHUNK_B, CHUNK)
    h0 = features[:, :HALF]
    h1 = features[:, HALF:]
    zeros = jnp.zeros((N_NODES, HALF), jnp.float32)
    b1r = b1.reshape(1, HID)
    b_outr = b_out.reshape(1, HID)
    for layer in range(3):
        agg0, agg1 = _sc_aggregate(h0, h1, src, dst, zeros)
        if layer < 2:
            h0, h1 = _tc_layer(agg0, agg1, W1, b1r)
    return _tc_final(agg0, agg1, W1, b1r, W_out, b_outr)
